# baseline (speedup 1.0000x reference)
import functools

import jax
import jax.numpy as jnp
from jax import lax
from jax.experimental import pallas as pl
from jax.experimental.pallas import tpu as pltpu
from jax.experimental.pallas import tpu_sc as plsc

N = 10000
D = 20
DP = 32
NP = 10240
E = 160000
NC, NS = 2, 16
NW = NC * NS
RNG = NP // NW
BE = 8000
NB = E // BE
CQ = 16
DRAIN = 128


CAP = 4096
SLABE = (E // CAP + 2) * CAP


def _sc_build_body(src_hbm, dst_hbm, slab_s_hbm, slab_d_hbm, cnt_hbm,
                   src_blk, dst_blk, stage_s, stage_d, cnt_v):
    c = lax.axis_index("c")
    s = lax.axis_index("s")
    wid = s * NC + c
    lo = wid * RNG
    zi = jnp.zeros((CQ,), jnp.int32)

    @pl.loop(0, (CAP + CQ) // CQ)
    def _(k):
        stage_s[pl.ds(k * CQ, CQ)] = zi
        stage_d[pl.ds(k * CQ, CQ)] = zi

    def scan_block(b, carry):
        pltpu.sync_copy(src_hbm.at[pl.ds(b * BE, BE)], src_blk)
        pltpu.sync_copy(dst_hbm.at[pl.ds(b * BE, BE)], dst_blk)

        def step(k, carry):
            off, fl = carry
            dv = dst_blk[pl.ds(k * CQ, CQ)]
            sv = src_blk[pl.ds(k * CQ, CQ)]
            mask = (dv >= lo) & (dv < lo + RNG)
            mi = mask.astype(jnp.int32)
            csum = plsc.cumsum(mi)
            pos = off + csum - mi
            plsc.store_scatter(stage_s, [pos], sv, mask=mask)
            plsc.store_scatter(stage_d, [pos], dv - lo, mask=mask)
            off = off + csum[CQ - 1]

            def flush(carry):
                off, fl = carry
                pltpu.sync_copy(stage_s.at[pl.ds(0, CAP)],
                                slab_s_hbm.at[wid, pl.ds(fl * CAP, CAP)])
                pltpu.sync_copy(stage_d.at[pl.ds(0, CAP)],
                                slab_d_hbm.at[wid, pl.ds(fl * CAP, CAP)])
                stage_s[pl.ds(0, CQ)] = stage_s[pl.ds(CAP, CQ)]
                stage_d[pl.ds(0, CQ)] = stage_d[pl.ds(CAP, CQ)]
                return (off - CAP, fl + 1)

            return lax.cond(off >= CAP, flush, lambda cc: cc, (off, fl))

        return lax.fori_loop(0, BE // CQ, step, carry)

    off, fl = lax.fori_loop(0, NB, scan_block,
                            (jnp.int32(0), jnp.int32(0)))
    pltpu.sync_copy(stage_s.at[pl.ds(0, CAP)],
                    slab_s_hbm.at[wid, pl.ds(fl * CAP, CAP)])
    pltpu.sync_copy(stage_d.at[pl.ds(0, CAP)],
                    slab_d_hbm.at[wid, pl.ds(fl * CAP, CAP)])
    cnt_v[pl.ds(0, CQ)] = jnp.full((CQ,), fl * CAP + off, jnp.int32)
    pltpu.sync_copy(cnt_v, cnt_hbm.at[wid])


@functools.lru_cache(maxsize=None)
def _sc_build():
    return pl.kernel(
        _sc_build_body,
        out_type=[jax.ShapeDtypeStruct((NW, SLABE), jnp.int32),
                  jax.ShapeDtypeStruct((NW, SLABE), jnp.int32),
                  jax.ShapeDtypeStruct((NW, CQ), jnp.int32)],
        mesh=plsc.VectorSubcoreMesh(core_axis_name="c", subcore_axis_name="s",
                                    num_cores=NC, num_subcores=NS),
        scratch_types=[
            pltpu.VMEM((BE,), jnp.int32),
            pltpu.VMEM((BE,), jnp.int32),
            pltpu.VMEM((CAP + CQ,), jnp.int32),
            pltpu.VMEM((CAP + CQ,), jnp.int32),
            pltpu.VMEM((CQ,), jnp.int32),
        ],
        compiler_params=pltpu.CompilerParams(use_tc_tiling_on_sc=False,
                                             needs_layout_passes=False),
    )


NSL = 4
SUBR = NP // NS


def _sc_apply_body(hm_hbm, slab_s_hbm, slab_d_hbm, cnt_hbm, out_hbm,
                   sidx, didx, rows_v, acc, cnt_v, hm_sp, gsem):
    c = lax.axis_index("c")
    s = lax.axis_index("s")
    wid = s * NC + c
    lo = wid * RNG

    pltpu.sync_copy(hm_hbm.at[pl.ds(s * SUBR, SUBR)],
                    hm_sp.at[pl.ds(s * SUBR, SUBR)])
    pltpu.sync_copy(cnt_hbm.at[wid], cnt_v)
    cnt = cnt_v[pl.ds(0, CQ)][0]
    zz = jnp.zeros((CQ,), jnp.float32)

    @pl.loop(0, RNG)
    def _(r):
        acc[r, 0:16] = zz
        acc[r, 16:32] = zz

    plsc.subcore_barrier()

    def load_start(g, b):
        pltpu.sync_copy(slab_s_hbm.at[wid, pl.ds(g * DRAIN, DRAIN)],
                        sidx.at[b, pl.ds(0, DRAIN)])
        pltpu.sync_copy(slab_d_hbm.at[wid, pl.ds(g * DRAIN, DRAIN)],
                        didx.at[b, pl.ds(0, DRAIN)])
        pltpu.async_copy(hm_sp.at[sidx.at[b, pl.ds(0, DRAIN)]],
                         rows_v.at[b], gsem.at[b])

    def wait_gather(b):
        pltpu.make_async_copy(hm_sp.at[sidx.at[b, pl.ds(0, DRAIN)]],
                              rows_v.at[b], gsem.at[b]).wait()

    ntot = cnt // DRAIN + 1
    ntot4 = (ntot + NSL - 1) // NSL * NSL
    for i in range(NSL - 1):
        if i == 0:
            load_start(0, 0)
        else:
            @pl.when(ntot > i)
            def _():
                load_start(i, i)

    @pl.loop(0, ntot4, step=NSL)
    def _(g):
        for b in range(NSL):
            gg = g + b

            @pl.when(gg < ntot)
            def _():
                @pl.when(gg + NSL - 1 < ntot)
                def _():
                    load_start(gg + NSL - 1, (b + NSL - 1) % NSL)
                wait_gather(b)
                nh = jnp.minimum(DRAIN, cnt - gg * DRAIN)

                @pl.loop(0, nh)
                def _(k):
                    dl = didx[b, pl.ds(k, CQ)][0]
                    acc[dl, 0:16] += rows_v[b, k, 0:16]
                    acc[dl, 16:32] += rows_v[b, k, 16:32]

    pltpu.sync_copy(acc, out_hbm.at[pl.ds(lo, RNG)])


@functools.lru_cache(maxsize=None)
def _sc_apply():
    return pl.kernel(
        _sc_apply_body,
        out_type=jax.ShapeDtypeStruct((NP, DP), jnp.float32),
        mesh=plsc.VectorSubcoreMesh(core_axis_name="c", subcore_axis_name="s",
                                    num_cores=NC, num_subcores=NS),
        scratch_types=[
            pltpu.VMEM((NSL, DRAIN + CQ), jnp.int32),
            pltpu.VMEM((NSL, DRAIN + CQ), jnp.int32),
            pltpu.VMEM((NSL, DRAIN, DP), jnp.float32),
            pltpu.VMEM((RNG, DP), jnp.float32),
            pltpu.VMEM((CQ,), jnp.int32),
            pltpu.VMEM_SHARED((NP, DP), jnp.float32),
            pltpu.SemaphoreType.DMA((NSL,)),
        ],
        compiler_params=pltpu.CompilerParams(use_tc_tiling_on_sc=False,
                                             needs_layout_passes=False),
    )


_BR = 2048


def _msg_body(h_ref, w_ref, b_ref, out_ref):
    m = lax.dot_general(h_ref[...], w_ref[0], (((1,), (1,)), ((), ())))
    m = m + b_ref[0:1]
    lane = lax.broadcasted_iota(jnp.int32, m.shape, 1)
    out_ref[...] = jnp.where(lane == D, 1.0, m)


_msg_call = pl.pallas_call(
    _msg_body,
    grid=(NP // _BR,),
    in_specs=[
        pl.BlockSpec((_BR, DP), lambda i: (i, 0)),
        pl.BlockSpec((7, DP, DP), lambda i: (0, 0, 0)),
        pl.BlockSpec((7, DP), lambda i: (0, 0)),
    ],
    out_specs=pl.BlockSpec((_BR, DP), lambda i: (i, 0)),
    out_shape=jax.ShapeDtypeStruct((NP, DP), jnp.float32),
)


def _gru_body(s_ref, hm_ref, h_ref, w_ref, b_ref, out_ref):
    h = h_ref[...]
    a = s_ref[...] + hm_ref[...]
    a = a + a[:, 20:21] * b_ref[0:1]
    dotT = lambda x, w: lax.dot_general(x, w, (((1,), (1,)), ((), ())))
    gi_r = dotT(a, w_ref[1]) + b_ref[1:2]
    gi_z = dotT(a, w_ref[2]) + b_ref[2:3]
    gi_n = dotT(a, w_ref[3]) + b_ref[3:4]
    gh_r = dotT(h, w_ref[4]) + b_ref[4:5]
    gh_z = dotT(h, w_ref[5]) + b_ref[5:6]
    gh_n = dotT(h, w_ref[6]) + b_ref[6:7]
    r = jax.nn.sigmoid(gi_r + gh_r)
    z = jax.nn.sigmoid(gi_z + gh_z)
    n = jnp.tanh(gi_n + r * gh_n)
    hn = jax.nn.relu((1.0 - z) * n + z * h)
    lane = lax.broadcasted_iota(jnp.int32, hn.shape, 1)
    out_ref[...] = jnp.where(lane < D, hn, 0.0)


_gru_call = pl.pallas_call(
    _gru_body,
    grid=(NP // _BR,),
    in_specs=[
        pl.BlockSpec((_BR, DP), lambda i: (i, 0)),
        pl.BlockSpec((_BR, DP), lambda i: (i, 0)),
        pl.BlockSpec((_BR, DP), lambda i: (i, 0)),
        pl.BlockSpec((7, DP, DP), lambda i: (0, 0, 0)),
        pl.BlockSpec((7, DP), lambda i: (0, 0)),
    ],
    out_specs=pl.BlockSpec((_BR, DP), lambda i: (i, 0)),
    out_shape=jax.ShapeDtypeStruct((NP, DP), jnp.float32),
)


def _mlp_body(h_ref, w_ref, b_ref, out_ref):
    dotT = lambda x, w: lax.dot_general(x, w, (((1,), (1,)), ((), ())))
    u = h_ref[...]
    u = jax.nn.relu(dotT(u, w_ref[0]) + b_ref[0:1])
    u = jax.nn.relu(dotT(u, w_ref[1]) + b_ref[1:2])
    u = jax.nn.relu(dotT(u, w_ref[2]) + b_ref[2:3])
    out_ref[...] = u


_mlp_call = pl.pallas_call(
    _mlp_body,
    grid=(NP // _BR,),
    in_specs=[
        pl.BlockSpec((_BR, DP), lambda i: (i, 0)),
        pl.BlockSpec((3, DP, DP), lambda i: (0, 0, 0)),
        pl.BlockSpec((3, DP), lambda i: (0, 0)),
    ],
    out_specs=pl.BlockSpec((_BR, DP), lambda i: (i, 0)),
    out_shape=jax.ShapeDtypeStruct((NP, DP), jnp.float32),
)

_BM = 400


def _gram_body(ui_ref, uall_ref, out_ref):
    out_ref[...] = lax.dot_general(ui_ref[...], uall_ref[:N, :],
                                   (((1,), (1,)), ((), ())))


_gram_call = pl.pallas_call(
    _gram_body,
    grid=(N // _BM,),
    in_specs=[
        pl.BlockSpec((_BM, DP), lambda i: (i, 0)),
        pl.BlockSpec((NP, DP), lambda i: (0, 0)),
    ],
    out_specs=pl.BlockSpec((_BM, N), lambda i: (i, 0)),
    out_shape=jax.ShapeDtypeStruct((N, N), jnp.float32),
)


def kernel(x, edge_index, edge_type, W_msg, b_msg, W_ih, W_hh, b_ih, b_hh,
           W1, b1, W2, b2, W3, b3):
    f32 = jnp.float32
    n_conv = W_msg.shape[0]
    src = edge_index[0]
    dst = edge_index[1]

    h = jnp.zeros((NP, DP), f32)
    h = h.at[:N, :x.shape[1]].set(x.astype(f32))

    def padw(w):
        return jnp.zeros((DP, DP), f32).at[:w.shape[0], :w.shape[1]].set(w)

    def padb(b):
        return jnp.zeros((DP,), f32).at[:b.shape[0]].set(b)

    wl, bl = [], []
    for i in range(n_conv):
        ihr, ihz, ihn = W_ih[i, :D], W_ih[i, D:2 * D], W_ih[i, 2 * D:]
        hhr, hhz, hhn = W_hh[i, :D], W_hh[i, D:2 * D], W_hh[i, 2 * D:]
        wl.append(jnp.stack([padw(W_msg[i]), padw(ihr), padw(ihz), padw(ihn),
                             padw(hhr), padw(hhz), padw(hhn)]))
        bl.append(jnp.stack([padb(b_msg[i]),
                             padb(b_ih[i, :D]), padb(b_ih[i, D:2 * D]),
                             padb(b_ih[i, 2 * D:]),
                             padb(b_hh[i, :D]), padb(b_hh[i, D:2 * D]),
                             padb(b_hh[i, 2 * D:])]))
    wmlp = jnp.stack([padw(W1), padw(W2), padw(W3)])
    bmlp = jnp.stack([padb(b1), padb(b2), padb(b3)])

    slab_s, slab_d, cnt = _sc_build()(src, dst)
    for i in range(n_conv):
        hm = _msg_call(h, wl[i], bl[i])
        s = _sc_apply()(hm, slab_s, slab_d, cnt)
        h = _gru_call(s, hm, h, wl[i], bl[i])

    u = _mlp_call(h, wmlp, bmlp)
    g = _gram_call(u, u)
    return g[None]


# speedup vs baseline: 1.1157x; 1.1157x over previous
import functools

import jax
import jax.numpy as jnp
from jax import lax
from jax.experimental import pallas as pl
from jax.experimental.pallas import tpu as pltpu
from jax.experimental.pallas import tpu_sc as plsc

N = 10000
D = 20
DP = 32
NP = 10240
E = 160000
NC, NS = 2, 16
NW = NC * NS
RNG = NP // NW
BE = 8000
NB = E // BE
CQ = 16
DRAIN = 128


CAP = 4096
SLABE = (E // CAP + 2) * CAP


def _sc_build_body(src_hbm, dst_hbm, slab_s_hbm, slab_d_hbm, cnt_hbm,
                   src_blk, dst_blk, stage_s, stage_d, cnt_v):
    c = lax.axis_index("c")
    s = lax.axis_index("s")
    wid = s * NC + c
    lo = wid * RNG
    zi = jnp.zeros((CQ,), jnp.int32)

    @pl.loop(0, (CAP + CQ) // CQ)
    def _(k):
        stage_s[pl.ds(k * CQ, CQ)] = zi
        stage_d[pl.ds(k * CQ, CQ)] = zi

    def scan_block(b, carry):
        pltpu.sync_copy(src_hbm.at[pl.ds(b * BE, BE)], src_blk)
        pltpu.sync_copy(dst_hbm.at[pl.ds(b * BE, BE)], dst_blk)

        def step(k, carry):
            off, fl = carry
            dv = dst_blk[pl.ds(k * CQ, CQ)]
            sv = src_blk[pl.ds(k * CQ, CQ)]
            mask = (dv >= lo) & (dv < lo + RNG)
            mi = mask.astype(jnp.int32)
            csum = plsc.cumsum(mi)
            pos = off + csum - mi
            plsc.store_scatter(stage_s, [pos], sv, mask=mask)
            plsc.store_scatter(stage_d, [pos], dv - lo, mask=mask)
            off = off + csum[CQ - 1]

            def flush(carry):
                off, fl = carry
                pltpu.sync_copy(stage_s.at[pl.ds(0, CAP)],
                                slab_s_hbm.at[wid, pl.ds(fl * CAP, CAP)])
                pltpu.sync_copy(stage_d.at[pl.ds(0, CAP)],
                                slab_d_hbm.at[wid, pl.ds(fl * CAP, CAP)])
                stage_s[pl.ds(0, CQ)] = stage_s[pl.ds(CAP, CQ)]
                stage_d[pl.ds(0, CQ)] = stage_d[pl.ds(CAP, CQ)]
                return (off - CAP, fl + 1)

            return lax.cond(off >= CAP, flush, lambda cc: cc, (off, fl))

        return lax.fori_loop(0, BE // CQ, step, carry)

    off, fl = lax.fori_loop(0, NB, scan_block,
                            (jnp.int32(0), jnp.int32(0)))
    pltpu.sync_copy(stage_s.at[pl.ds(0, CAP)],
                    slab_s_hbm.at[wid, pl.ds(fl * CAP, CAP)])
    pltpu.sync_copy(stage_d.at[pl.ds(0, CAP)],
                    slab_d_hbm.at[wid, pl.ds(fl * CAP, CAP)])
    cnt_v[pl.ds(0, CQ)] = jnp.full((CQ,), fl * CAP + off, jnp.int32)
    pltpu.sync_copy(cnt_v, cnt_hbm.at[wid])


@functools.lru_cache(maxsize=None)
def _sc_build():
    return pl.kernel(
        _sc_build_body,
        out_type=[jax.ShapeDtypeStruct((NW, SLABE), jnp.int32),
                  jax.ShapeDtypeStruct((NW, SLABE), jnp.int32),
                  jax.ShapeDtypeStruct((NW, CQ), jnp.int32)],
        mesh=plsc.VectorSubcoreMesh(core_axis_name="c", subcore_axis_name="s",
                                    num_cores=NC, num_subcores=NS),
        scratch_types=[
            pltpu.VMEM((BE,), jnp.int32),
            pltpu.VMEM((BE,), jnp.int32),
            pltpu.VMEM((CAP + CQ,), jnp.int32),
            pltpu.VMEM((CAP + CQ,), jnp.int32),
            pltpu.VMEM((CQ,), jnp.int32),
        ],
        compiler_params=pltpu.CompilerParams(use_tc_tiling_on_sc=False,
                                             needs_layout_passes=False),
    )


NSL = 4
SUBR = NP // NS


def _sc_apply_body(hm_hbm, slab_s_hbm, slab_d_hbm, cnt_hbm, out_hbm,
                   sidx, didx, rows_v, acc, cnt_v, hm_sp, gsem, rsem):
    c = lax.axis_index("c")
    s = lax.axis_index("s")
    wid = s * NC + c
    lo = wid * RNG

    pltpu.sync_copy(hm_hbm.at[pl.ds(s * SUBR, SUBR)],
                    hm_sp.at[pl.ds(s * SUBR, SUBR)])
    pltpu.sync_copy(cnt_hbm.at[wid], cnt_v)
    cnt = cnt_v[pl.ds(0, CQ)][0]
    zz = jnp.zeros((CQ,), jnp.float32)

    @pl.loop(0, RNG)
    def _(r):
        acc[r, 0:16] = zz
        acc[r, 16:32] = zz

    plsc.subcore_barrier()

    def load_start(g, b):
        pltpu.async_copy(slab_s_hbm.at[wid, pl.ds(g * DRAIN, DRAIN)],
                         sidx.at[b, pl.ds(0, DRAIN)], gsem.at[b])
        pltpu.async_copy(slab_d_hbm.at[wid, pl.ds(g * DRAIN, DRAIN)],
                         didx.at[b, pl.ds(0, DRAIN)], gsem.at[b])

    def wait_gather(g, b):
        pltpu.make_async_copy(slab_s_hbm.at[wid, pl.ds(g * DRAIN, DRAIN)],
                              sidx.at[b, pl.ds(0, DRAIN)], gsem.at[b]).wait()
        pltpu.make_async_copy(slab_d_hbm.at[wid, pl.ds(g * DRAIN, DRAIN)],
                              didx.at[b, pl.ds(0, DRAIN)], gsem.at[b]).wait()
        pltpu.async_copy(hm_sp.at[sidx.at[b, pl.ds(0, DRAIN)]],
                         rows_v.at[b], rsem).wait()

    ntot = cnt // DRAIN + 1
    ntot4 = (ntot + NSL - 1) // NSL * NSL
    for i in range(NSL - 1):
        if i == 0:
            load_start(0, 0)
        else:
            @pl.when(ntot > i)
            def _():
                load_start(i, i)

    @pl.loop(0, ntot4, step=NSL)
    def _(g):
        for b in range(NSL):
            gg = g + b

            @pl.when(gg < ntot)
            def _():
                @pl.when(gg + NSL - 1 < ntot)
                def _():
                    load_start(gg + NSL - 1, (b + NSL - 1) % NSL)
                wait_gather(gg, b)
                nh = jnp.minimum(DRAIN, cnt - gg * DRAIN)

                @pl.loop(0, nh)
                def _(k):
                    dl = didx[b, pl.ds(k, CQ)][0]
                    acc[dl, 0:16] += rows_v[b, k, 0:16]
                    acc[dl, 16:32] += rows_v[b, k, 16:32]

    pltpu.sync_copy(acc, out_hbm.at[pl.ds(lo, RNG)])


@functools.lru_cache(maxsize=None)
def _sc_apply():
    return pl.kernel(
        _sc_apply_body,
        out_type=jax.ShapeDtypeStruct((NP, DP), jnp.float32),
        mesh=plsc.VectorSubcoreMesh(core_axis_name="c", subcore_axis_name="s",
                                    num_cores=NC, num_subcores=NS),
        scratch_types=[
            pltpu.VMEM((NSL, DRAIN + CQ), jnp.int32),
            pltpu.VMEM((NSL, DRAIN + CQ), jnp.int32),
            pltpu.VMEM((NSL, DRAIN, DP), jnp.float32),
            pltpu.VMEM((RNG, DP), jnp.float32),
            pltpu.VMEM((CQ,), jnp.int32),
            pltpu.VMEM_SHARED((NP, DP), jnp.float32),
            pltpu.SemaphoreType.DMA((NSL,)),
            pltpu.SemaphoreType.DMA,
        ],
        compiler_params=pltpu.CompilerParams(use_tc_tiling_on_sc=False,
                                             needs_layout_passes=False),
    )


_BR = 2048


def _msg_body(h_ref, w_ref, b_ref, out_ref):
    m = lax.dot_general(h_ref[...], w_ref[0], (((1,), (1,)), ((), ())))
    m = m + b_ref[0:1]
    lane = lax.broadcasted_iota(jnp.int32, m.shape, 1)
    out_ref[...] = jnp.where(lane == D, 1.0, m)


_msg_call = pl.pallas_call(
    _msg_body,
    grid=(NP // _BR,),
    in_specs=[
        pl.BlockSpec((_BR, DP), lambda i: (i, 0)),
        pl.BlockSpec((7, DP, DP), lambda i: (0, 0, 0)),
        pl.BlockSpec((7, DP), lambda i: (0, 0)),
    ],
    out_specs=pl.BlockSpec((_BR, DP), lambda i: (i, 0)),
    out_shape=jax.ShapeDtypeStruct((NP, DP), jnp.float32),
)


def _gru_body(s_ref, hm_ref, h_ref, w_ref, b_ref, out_ref):
    h = h_ref[...]
    a = s_ref[...] + hm_ref[...]
    a = a + a[:, 20:21] * b_ref[0:1]
    dotT = lambda x, w: lax.dot_general(x, w, (((1,), (1,)), ((), ())))
    gi_r = dotT(a, w_ref[1]) + b_ref[1:2]
    gi_z = dotT(a, w_ref[2]) + b_ref[2:3]
    gi_n = dotT(a, w_ref[3]) + b_ref[3:4]
    gh_r = dotT(h, w_ref[4]) + b_ref[4:5]
    gh_z = dotT(h, w_ref[5]) + b_ref[5:6]
    gh_n = dotT(h, w_ref[6]) + b_ref[6:7]
    r = jax.nn.sigmoid(gi_r + gh_r)
    z = jax.nn.sigmoid(gi_z + gh_z)
    n = jnp.tanh(gi_n + r * gh_n)
    hn = jax.nn.relu((1.0 - z) * n + z * h)
    lane = lax.broadcasted_iota(jnp.int32, hn.shape, 1)
    out_ref[...] = jnp.where(lane < D, hn, 0.0)


_gru_call = pl.pallas_call(
    _gru_body,
    grid=(NP // _BR,),
    in_specs=[
        pl.BlockSpec((_BR, DP), lambda i: (i, 0)),
        pl.BlockSpec((_BR, DP), lambda i: (i, 0)),
        pl.BlockSpec((_BR, DP), lambda i: (i, 0)),
        pl.BlockSpec((7, DP, DP), lambda i: (0, 0, 0)),
        pl.BlockSpec((7, DP), lambda i: (0, 0)),
    ],
    out_specs=pl.BlockSpec((_BR, DP), lambda i: (i, 0)),
    out_shape=jax.ShapeDtypeStruct((NP, DP), jnp.float32),
)


def _mlp_body(h_ref, w_ref, b_ref, out_ref):
    dotT = lambda x, w: lax.dot_general(x, w, (((1,), (1,)), ((), ())))
    u = h_ref[...]
    u = jax.nn.relu(dotT(u, w_ref[0]) + b_ref[0:1])
    u = jax.nn.relu(dotT(u, w_ref[1]) + b_ref[1:2])
    u = jax.nn.relu(dotT(u, w_ref[2]) + b_ref[2:3])
    out_ref[...] = u


_mlp_call = pl.pallas_call(
    _mlp_body,
    grid=(NP // _BR,),
    in_specs=[
        pl.BlockSpec((_BR, DP), lambda i: (i, 0)),
        pl.BlockSpec((3, DP, DP), lambda i: (0, 0, 0)),
        pl.BlockSpec((3, DP), lambda i: (0, 0)),
    ],
    out_specs=pl.BlockSpec((_BR, DP), lambda i: (i, 0)),
    out_shape=jax.ShapeDtypeStruct((NP, DP), jnp.float32),
)

_BM = 400


def _gram_body(ui_ref, uall_ref, out_ref):
    out_ref[...] = lax.dot_general(ui_ref[...], uall_ref[:N, :],
                                   (((1,), (1,)), ((), ())))


_gram_call = pl.pallas_call(
    _gram_body,
    grid=(N // _BM,),
    in_specs=[
        pl.BlockSpec((_BM, DP), lambda i: (i, 0)),
        pl.BlockSpec((NP, DP), lambda i: (0, 0)),
    ],
    out_specs=pl.BlockSpec((_BM, N), lambda i: (i, 0)),
    out_shape=jax.ShapeDtypeStruct((N, N), jnp.float32),
)


def kernel(x, edge_index, edge_type, W_msg, b_msg, W_ih, W_hh, b_ih, b_hh,
           W1, b1, W2, b2, W3, b3):
    f32 = jnp.float32
    n_conv = W_msg.shape[0]
    src = edge_index[0]
    dst = edge_index[1]

    h = jnp.zeros((NP, DP), f32)
    h = h.at[:N, :x.shape[1]].set(x.astype(f32))

    def padw(w):
        return jnp.zeros((DP, DP), f32).at[:w.shape[0], :w.shape[1]].set(w)

    def padb(b):
        return jnp.zeros((DP,), f32).at[:b.shape[0]].set(b)

    wl, bl = [], []
    for i in range(n_conv):
        ihr, ihz, ihn = W_ih[i, :D], W_ih[i, D:2 * D], W_ih[i, 2 * D:]
        hhr, hhz, hhn = W_hh[i, :D], W_hh[i, D:2 * D], W_hh[i, 2 * D:]
        wl.append(jnp.stack([padw(W_msg[i]), padw(ihr), padw(ihz), padw(ihn),
                             padw(hhr), padw(hhz), padw(hhn)]))
        bl.append(jnp.stack([padb(b_msg[i]),
                             padb(b_ih[i, :D]), padb(b_ih[i, D:2 * D]),
                             padb(b_ih[i, 2 * D:]),
                             padb(b_hh[i, :D]), padb(b_hh[i, D:2 * D]),
                             padb(b_hh[i, 2 * D:])]))
    wmlp = jnp.stack([padw(W1), padw(W2), padw(W3)])
    bmlp = jnp.stack([padb(b1), padb(b2), padb(b3)])

    slab_s, slab_d, cnt = _sc_build()(src, dst)
    for i in range(n_conv):
        hm = _msg_call(h, wl[i], bl[i])
        s = _sc_apply()(hm, slab_s, slab_d, cnt)
        h = _gru_call(s, hm, h, wl[i], bl[i])

    u = _mlp_call(h, wmlp, bmlp)
    g = _gram_call(u, u)
    return g[None]
